# baseline (speedup 1.0000x reference)
import functools

import jax
import jax.numpy as jnp
from jax.experimental import pallas as pl
from jax.experimental.pallas import tpu as pltpu
from jax.experimental.pallas import tpu_sc as plsc

_GATHER = 128
_K = 4
_CHUNK = _GATHER * _K


def kernel(pos, pos_embeddings):
    batch, hist = pos.shape
    _, dim = pos_embeddings.shape
    num_idx = batch * hist
    idx = pos.reshape(1, num_idx)

    mesh = plsc.VectorSubcoreMesh(core_axis_name="core", subcore_axis_name="subcore")

    @functools.partial(
        pl.kernel,
        out_type=jax.ShapeDtypeStruct((num_idx, dim), pos_embeddings.dtype),
        mesh=mesh,
        scratch_types=[pltpu.SemaphoreType.DMA],
        compiler_params=pltpu.CompilerParams(use_tc_tiling_on_sc=False),
    )
    def gather_kernel(table_hbm, i_hbm, o_hbm, sem):
        def body(i_vmem, o_vmem):
            copies = []
            for j in range(_K):
                copies.append(
                    pltpu.async_copy(
                        table_hbm.at[i_vmem.at[0, pl.ds(j * _GATHER, _GATHER)]],
                        o_vmem.at[pl.ds(j * _GATHER, _GATHER), :],
                        sem,
                    )
                )
            for c in copies:
                c.wait()

        pltpu.emit_pipeline(
            body,
            grid=(num_idx // _CHUNK,),
            in_specs=[pl.BlockSpec((1, _CHUNK), lambda i: (0, i))],
            out_specs=[pl.BlockSpec((_CHUNK, dim), lambda i: (i, 0))],
            core_axis_name=("core", "subcore"),
            dimension_semantics=(pltpu.PARALLEL,),
        )(i_hbm, o_hbm)

    out = gather_kernel(pos_embeddings, idx)
    return out.reshape(batch, hist, dim)


# speedup vs baseline: 2.2700x; 2.2700x over previous
import functools

import jax
import jax.numpy as jnp
from jax import lax
from jax.experimental import pallas as pl
from jax.experimental.pallas import tpu as pltpu
from jax.experimental.pallas import tpu_sc as plsc

_WINDOW = 128


def kernel(pos, pos_embeddings):
    batch, hist = pos.shape
    rows, dim = pos_embeddings.shape
    num_idx = batch * hist
    idx = pos.reshape(1, num_idx)

    mesh = plsc.VectorSubcoreMesh(core_axis_name="core", subcore_axis_name="subcore")

    @functools.partial(
        pl.kernel,
        out_type=jax.ShapeDtypeStruct((num_idx, dim), pos_embeddings.dtype),
        mesh=mesh,
        scratch_types=[pltpu.VMEM_SHARED((rows, dim), pos_embeddings.dtype)],
        compiler_params=pltpu.CompilerParams(use_tc_tiling_on_sc=False),
    )
    def gather_kernel(table_hbm, i_hbm, o_hbm, table_sp):
        @pl.when(lax.axis_index("subcore") == 0)
        def _():
            pltpu.sync_copy(table_hbm, table_sp)

        plsc.subcore_barrier()

        def body(i_vmem, o_vmem):
            pltpu.sync_copy(table_sp.at[i_vmem.at[0]], o_vmem)

        pltpu.emit_pipeline(
            body,
            grid=(num_idx // _WINDOW,),
            in_specs=[pl.BlockSpec((1, _WINDOW), lambda i: (0, i))],
            out_specs=[pl.BlockSpec((_WINDOW, dim), lambda i: (i, 0))],
            core_axis_name=("core", "subcore"),
            dimension_semantics=(pltpu.PARALLEL,),
        )(i_hbm, o_hbm)

    out = gather_kernel(pos_embeddings, idx)
    return out.reshape(batch, hist, dim)
